# Initial kernel scaffold; baseline (speedup 1.0000x reference)
import jax
import jax.numpy as jnp
from jax.experimental import pallas as pl


def kernel(input_ids, token_type_ids, tok_table, seg_table, pe):
    raise NotImplementedError("write your pallas kernel here")


# speedup vs baseline: 7.2986x; 7.2986x over previous
import jax
import jax.numpy as jnp
from jax.experimental import pallas as pl
from jax.experimental.pallas import tpu as pltpu

_CHUNK = 128


def _embed_kernel(ids_ref, tt_ref, seg_ref, pe_ref, tok_hbm_ref, out_ref,
                  tok_buf, sems):
    b = pl.program_id(0)
    S, D = pe_ref.shape
    n_chunks = S // _CHUNK

    def issue_chunk(c, slot):
        base = c * _CHUNK
        for r in range(_CHUNK):
            pltpu.make_async_copy(
                tok_hbm_ref.at[pl.ds(ids_ref[b, base + r], 1), :],
                tok_buf.at[slot, pl.ds(r, 1), :],
                sems.at[slot]).start()

    def wait_chunk(slot):
        pltpu.make_async_copy(
            tok_hbm_ref.at[pl.ds(0, _CHUNK), :],
            tok_buf.at[slot],
            sems.at[slot]).wait()

    issue_chunk(0, 0)
    for c in range(n_chunks):
        slot = c & 1
        if c + 1 < n_chunks:
            issue_chunk(c + 1, 1 - slot)
        wait_chunk(slot)

        off = c * _CHUNK
        tok = tok_buf[slot]
        tt = tt_ref[0, pl.ds(off, _CHUNK), :]
        seg = jnp.where(tt == 0, seg_ref[0:1, :], seg_ref[1:2, :])
        out_ref[0, pl.ds(off, _CHUNK), :] = tok + pe_ref[pl.ds(off, _CHUNK), :] + seg


def kernel(input_ids, token_type_ids, tok_table, seg_table, pe):
    B, S = input_ids.shape
    V, D = tok_table.shape
    T = seg_table.shape[0]

    if token_type_ids is None:
        token_type_ids = jnp.zeros_like(input_ids)
    ids = input_ids.astype(jnp.int32)
    tt_3d = token_type_ids.astype(jnp.int32).reshape(B, S, 1)

    grid_spec = pltpu.PrefetchScalarGridSpec(
        num_scalar_prefetch=1,
        grid=(B,),
        in_specs=[
            pl.BlockSpec((1, S, 1), lambda b, ids_ref: (b, 0, 0)),
            pl.BlockSpec((T, D), lambda b, ids_ref: (0, 0)),
            pl.BlockSpec((S, D), lambda b, ids_ref: (0, 0)),
            pl.BlockSpec(memory_space=pl.ANY),
        ],
        out_specs=pl.BlockSpec((1, S, D), lambda b, ids_ref: (b, 0, 0)),
        scratch_shapes=[
            pltpu.VMEM((2, _CHUNK, D), jnp.float32),
            pltpu.SemaphoreType.DMA((2,)),
        ],
    )
    return pl.pallas_call(
        _embed_kernel,
        out_shape=jax.ShapeDtypeStruct((B, S, D), jnp.float32),
        grid_spec=grid_spec,
        compiler_params=pltpu.CompilerParams(
            dimension_semantics=("parallel",),
            disable_bounds_checks=True,
        ),
    )(ids, tt_3d, seg_table, pe, tok_table)


# speedup vs baseline: 8.9514x; 1.2265x over previous
import jax
import jax.numpy as jnp
from jax.experimental import pallas as pl
from jax.experimental.pallas import tpu as pltpu

_CHUNK = 128


def _embed_kernel(ids_ref, tt_ref, seg_ref, pe_ref, tok_hbm_ref, out_ref,
                  tok_buf, sems):
    b = pl.program_id(0)
    S, D = pe_ref.shape
    n_chunks = S // _CHUNK

    def issue_chunk(c, slot):
        base = c * _CHUNK
        for r in range(_CHUNK):
            pltpu.make_async_copy(
                tok_hbm_ref.at[pl.ds(ids_ref[b, base + r], 1), :],
                tok_buf.at[slot, pl.ds(r, 1), :],
                sems.at[slot]).start()

    def wait_chunk(slot):
        pltpu.make_async_copy(
            tok_hbm_ref.at[pl.ds(0, _CHUNK), :],
            tok_buf.at[slot],
            sems.at[slot]).wait()

    for c in range(n_chunks):
        issue_chunk(c, c)
    for c in range(n_chunks):
        slot = c
        wait_chunk(slot)

        off = c * _CHUNK
        tok = tok_buf[slot]
        tt = tt_ref[0, pl.ds(off, _CHUNK), :]
        seg = jnp.where(tt == 0, seg_ref[0:1, :], seg_ref[1:2, :])
        out_ref[0, pl.ds(off, _CHUNK), :] = tok + pe_ref[pl.ds(off, _CHUNK), :] + seg


def kernel(input_ids, token_type_ids, tok_table, seg_table, pe):
    B, S = input_ids.shape
    V, D = tok_table.shape
    T = seg_table.shape[0]

    if token_type_ids is None:
        token_type_ids = jnp.zeros_like(input_ids)
    ids = input_ids.astype(jnp.int32)
    tt_3d = token_type_ids.astype(jnp.int32).reshape(B, S, 1)

    grid_spec = pltpu.PrefetchScalarGridSpec(
        num_scalar_prefetch=1,
        grid=(B,),
        in_specs=[
            pl.BlockSpec((1, S, 1), lambda b, ids_ref: (b, 0, 0)),
            pl.BlockSpec((T, D), lambda b, ids_ref: (0, 0)),
            pl.BlockSpec((S, D), lambda b, ids_ref: (0, 0)),
            pl.BlockSpec(memory_space=pl.ANY),
        ],
        out_specs=pl.BlockSpec((1, S, D), lambda b, ids_ref: (b, 0, 0)),
        scratch_shapes=[
            pltpu.VMEM((S // _CHUNK, _CHUNK, D), jnp.float32),
            pltpu.SemaphoreType.DMA((S // _CHUNK,)),
        ],
    )
    return pl.pallas_call(
        _embed_kernel,
        out_shape=jax.ShapeDtypeStruct((B, S, D), jnp.float32),
        grid_spec=grid_spec,
        compiler_params=pltpu.CompilerParams(
            dimension_semantics=("parallel",),
            disable_bounds_checks=True,
        ),
    )(ids, tt_3d, seg_table, pe, tok_table)
